# baseline (speedup 1.0000x reference)
import functools

import jax
import jax.numpy as jnp
from jax import lax
from jax.experimental import pallas as pl
from jax.experimental.pallas import tpu as pltpu
from jax.experimental.pallas import tpu_sc as plsc

NN = 10000
NP = 10240
EE = 320000
RR = 4
DD = 128
GG = NP // 128
GB = 16
NBLK = GG // GB

NC = 2
NS = 16
EPT = EE // NS
EPT32 = EE // (NC * NS)
KE = 4000
KE2 = 2000
CAP = 20480
NPH = 3584
NRNG = 3
RPT = NPH // NS

def _mesh():
    return plsc.VectorSubcoreMesh(
        core_axis_name="c", subcore_axis_name="s",
        num_cores=NC, num_subcores=NS)


def _sc_counts_body(src_hbm, dst_hbm, typ_hbm, cnt_hbm, epk_hbm,
                    sbuf, dbuf, tbuf, pbuf, acc):
    c = lax.axis_index("c")
    s = lax.axis_index("s")
    wid = s * NC + c
    z16 = jnp.zeros((16,), jnp.float32)
    ones = jnp.ones((16,), jnp.float32)

    def zero(i, _):
        acc[pl.ds(i * 16, 16)] = z16
        return 0
    lax.fori_loop(0, RR * NP // 16, zero, 0)

    def stage(st, _):
        base = wid * EPT32 + st * KE2
        pltpu.sync_copy(src_hbm.at[pl.ds(base, KE2)], sbuf)
        pltpu.sync_copy(dst_hbm.at[pl.ds(base, KE2)], dbuf)
        pltpu.sync_copy(typ_hbm.at[pl.ds(base, KE2)], tbuf)

        def chunk(i, _):
            s16 = sbuf[pl.ds(i * 16, 16)]
            d16 = dbuf[pl.ds(i * 16, 16)]
            t16 = tbuf[pl.ds(i * 16, 16)]
            plsc.addupdate_scatter(acc, [t16 * NP + d16], ones)
            pbuf[pl.ds(i * 16, 16)] = t16 * 268435456 + s16 * 16384 + d16
            return 0
        lax.fori_loop(0, KE2 // 16, chunk, 0)
        pltpu.sync_copy(pbuf, epk_hbm.at[pl.ds(base, KE2)])
        return 0
    lax.fori_loop(0, EPT32 // KE2, stage, 0)
    pltpu.sync_copy(acc, cnt_hbm.at[wid])


@functools.cache
def _sc_counts():
    return pl.kernel(
        _sc_counts_body,
        out_type=(jax.ShapeDtypeStruct((NC * NS, RR * NP), jnp.float32),
                  jax.ShapeDtypeStruct((EE,), jnp.int32)),
        mesh=_mesh(),
        scratch_types=[
            pltpu.VMEM((KE2,), jnp.int32),
            pltpu.VMEM((KE2,), jnp.int32),
            pltpu.VMEM((KE2,), jnp.int32),
            pltpu.VMEM((KE2,), jnp.int32),
            pltpu.VMEM((RR * NP,), jnp.float32),
        ],
        compiler_params=pltpu.CompilerParams(needs_layout_passes=False),
    )


def _sc_msg_body(tab_hbm, epk_hbm, out_hbm,
                 ebuf, prel, qbuf, gidx2, dbuf2, rbuf, zbuf, acc,
                 semG0, semG1, semG2, semG3, semS0, semS1, semS2, semS3):
    c = lax.axis_index("c")
    s = lax.axis_index("s")
    iota = lax.iota(jnp.int32, 16)
    z16 = jnp.zeros((16,), jnp.float32)

    pltpu.sync_copy(epk_hbm.at[pl.ds(s * EPT, EPT)], ebuf)

    def zrow(i, _):
        def zcol(k, _):
            zbuf[i, pl.ds(k * 16, 16)] = z16
            return 0
        lax.fori_loop(0, DD // 16, zcol, 0)
        return 0
    lax.fori_loop(0, 16, zrow, 0)

    def one_rel(p, _):
        r = 2 * c + p
        def rchunk(i, offr):
            p16 = ebuf[pl.ds(i * 16, 16)]
            m = (p16 // 268435456) == r
            plsc.store_compressed(prel.at[pl.ds(offr, 16)],
                                  p16 % 268435456, mask=m)
            return offr + jnp.sum(m.astype(jnp.int32))
        offr = lax.fori_loop(0, EPT // 16, rchunk, 0)
        prel[pl.ds(offr, 16)] = jnp.full((16,), 268435455, jnp.int32)
        nrc = (offr + 15) // 16

        def one_rng(h, _):
            lo = h * NPH
            def zacc(k, _):
                pltpu.sync_copy(zbuf, acc.at[pl.ds(s * RPT + k * 16, 16)])
                return 0
            lax.fori_loop(0, RPT // 16, zacc, 0)
            plsc.subcore_barrier()

            def qchunk(i, off):
                q16 = prel[pl.ds(i * 16, 16)]
                d16 = q16 % 16384
                s16 = q16 // 16384
                dl = d16 - lo
                m = (dl >= 0) & (dl < NPH)
                plsc.store_compressed(qbuf.at[pl.ds(off, 16)],
                                      (r * NP + s16) * 4096 + dl, mask=m)
                return off + jnp.sum(m.astype(jnp.int32))
            off = lax.fori_loop(0, nrc, qchunk, 0)

            def pad(k, _):
                qbuf[pl.ds(off + k * 16, 16)] = (
                    (r * NP + NN + (k % 4) * 16 + iota) * 4096
                    + (k % 4) * 16 + iota)
                return 0
            lax.fori_loop(0, 16, pad, 0)
            nbp = ((off + 255) // 256) * 2

            ngrp = (off + 255) // 256
            nbp = ngrp * 4

            def unpack(j, b):
                def cp16(k, _):
                    v = qbuf[pl.ds(j * 64 + k * 16, 16)]
                    gidx2[b, pl.ds(k * 16, 16)] = v // 4096
                    dbuf2[b, pl.ds(k * 16, 16)] = v % 4096
                    return 0
                lax.fori_loop(0, 4, cp16, 0)

            def fireG(j, b, sem):
                unpack(j, b)
                pltpu.async_copy(tab_hbm.at[gidx2.at[b]], rbuf.at[b], sem)

            def waitG(b, sem):
                pltpu.make_async_copy(
                    tab_hbm.at[gidx2.at[b]], rbuf.at[b], sem).wait()

            def fireS(b, sem):
                pltpu.async_copy(rbuf.at[b], acc.at[dbuf2.at[b]], sem,
                                 add=True)

            def waitS(b, sem):
                pltpu.make_async_copy(
                    rbuf.at[b], acc.at[dbuf2.at[b]], sem).wait()

            semG = (semG0, semG1, semG2, semG3)
            semS = (semS0, semS1, semS2, semS3)

            @pl.when(nbp > 0)
            def _prologue():
                fireG(0, 0, semG[0])
                fireG(1, 1, semG[1])

            def group(g, _):
                j0 = g * 4
                for t in range(4):
                    j = j0 + t
                    b = t
                    waitG(b, semG[b])
                    fireS(b, semS[b])
                    jn = j + 2
                    bn = (t + 2) % 4

                    @pl.when(jn < nbp)
                    def _():
                        @pl.when(jn >= 4)
                        def _():
                            waitS(bn, semS[bn])
                        fireG(jn, bn, semG[bn])
                return 0
            lax.fori_loop(0, ngrp, group, 0)

            @pl.when(nbp > 0)
            def _epilogue():
                waitS(0, semS[0])
                waitS(1, semS[1])
                waitS(2, semS[2])
                waitS(3, semS[3])
            plsc.subcore_barrier()

            grow = lo + s * RPT
            @pl.when(grow + RPT <= NP)
            def _writeback():
                pltpu.sync_copy(acc.at[pl.ds(s * RPT, RPT)],
                                out_hbm.at[pl.ds(r * NP + grow, RPT)])
            plsc.subcore_barrier()
            return 0
        lax.fori_loop(0, NRNG, one_rng, 0)
        return 0
    lax.fori_loop(0, 2, one_rel, 0)


@functools.cache
def _sc_msg():
    return pl.kernel(
        _sc_msg_body,
        out_type=jax.ShapeDtypeStruct((RR * NP, DD), jnp.float32),
        mesh=_mesh(),
        scratch_types=[
            pltpu.VMEM((EPT,), jnp.int32),
            pltpu.VMEM((CAP,), jnp.int32),
            pltpu.VMEM((CAP,), jnp.int32),
            pltpu.VMEM((4, 64), jnp.int32),
            pltpu.VMEM((4, 64), jnp.int32),
            pltpu.VMEM((4, 64, DD), jnp.float32),
            pltpu.VMEM((16, DD), jnp.float32),
            pltpu.MemorySpace.VMEM_SHARED((NPH, DD), jnp.float32),
            pltpu.SemaphoreType.DMA,
            pltpu.SemaphoreType.DMA,
            pltpu.SemaphoreType.DMA,
            pltpu.SemaphoreType.DMA,
            pltpu.SemaphoreType.DMA,
            pltpu.SemaphoreType.DMA,
            pltpu.SemaphoreType.DMA,
            pltpu.SemaphoreType.DMA,
        ],
        compiler_params=pltpu.CompilerParams(needs_layout_passes=False),
    )


def _tc1_body(cntp_ref, x_ref, w1_ref, dis_ref, t1_ref):
    deg = jnp.sum(cntp_ref[:, 0], axis=0) + 1.0
    d = lax.rsqrt(deg)
    dis_ref[0] = d
    h = jnp.dot(jnp.reshape(x_ref[...], (GB * 128, DD)), w1_ref[0],
                preferred_element_type=jnp.float32)
    t1_ref[0] = jnp.reshape(h, (GB, 128, DD)) * d[:, :, None]


def _tc2_body(m1_ref, t1_ref, dis_ref, b1_ref, w2_ref, t2_ref):
    d = dis_ref[0]
    pre = (m1_ref[0] + t1_ref[0]) * d[:, :, None] + b1_ref[0, 0][None, None, :]
    h = jnp.maximum(pre, 0.0)
    h2 = jnp.dot(jnp.reshape(h, (GB * 128, DD)), w2_ref[0],
                 preferred_element_type=jnp.float32)
    i = pl.program_id(1)
    nid3 = (i * GB * 128
            + lax.broadcasted_iota(jnp.int32, (GB, 128, DD), 0) * 128
            + lax.broadcasted_iota(jnp.int32, (GB, 128, DD), 1))
    t2 = jnp.reshape(h2, (GB, 128, DD)) * d[:, :, None]
    t2_ref[0] = jnp.where(nid3 < NN, t2, 0.0)


def _tc3_body(m2_ref, t2_ref, dis_ref, b2_ref, wa_ref, out_ref):
    d = dis_ref[...]
    emb = ((m2_ref[...] + t2_ref[...]) * d[..., None]
           + b2_ref[:, 0][:, None, None, :])
    wa = wa_ref[0]
    logits = jnp.sum(emb * wa[None, None, None, :], axis=-1)
    mx = jnp.max(logits, axis=0)
    ex = jnp.exp(logits - mx[None])
    att = ex / jnp.sum(ex, axis=0)[None]
    out_ref[...] = jnp.sum(emb * att[..., None], axis=0)


def _tc1(cntp, x3, W1):
    return pl.pallas_call(
        _tc1_body,
        grid=(RR, NBLK),
        in_specs=[
            pl.BlockSpec((NC * NS, 1, GB, 128), lambda r, i: (0, r, i, 0)),
            pl.BlockSpec((GB, 128, DD), lambda r, i: (i, 0, 0)),
            pl.BlockSpec((1, DD, DD), lambda r, i: (r, 0, 0)),
        ],
        out_specs=[
            pl.BlockSpec((1, GB, 128), lambda r, i: (r, i, 0)),
            pl.BlockSpec((1, GB, 128, DD), lambda r, i: (r, i, 0, 0)),
        ],
        out_shape=[
            jax.ShapeDtypeStruct((RR, GG, 128), jnp.float32),
            jax.ShapeDtypeStruct((RR, GG, 128, DD), jnp.float32),
        ],
    )(cntp, x3, W1)


def _tc2(m1, t1, dis, b1, W2):
    return pl.pallas_call(
        _tc2_body,
        grid=(RR, NBLK),
        in_specs=[
            pl.BlockSpec((1, GB, 128, DD), lambda r, i: (r, i, 0, 0)),
            pl.BlockSpec((1, GB, 128, DD), lambda r, i: (r, i, 0, 0)),
            pl.BlockSpec((1, GB, 128), lambda r, i: (r, i, 0)),
            pl.BlockSpec((1, 1, DD), lambda r, i: (r, 0, 0)),
            pl.BlockSpec((1, DD, DD), lambda r, i: (r, 0, 0)),
        ],
        out_specs=pl.BlockSpec((1, GB, 128, DD), lambda r, i: (r, i, 0, 0)),
        out_shape=jax.ShapeDtypeStruct((RR, GG, 128, DD), jnp.float32),
    )(m1, t1, dis, b1, W2)


def _tc3(m2, t2, dis, b2, wa):
    return pl.pallas_call(
        _tc3_body,
        grid=(NBLK,),
        in_specs=[
            pl.BlockSpec((RR, GB, 128, DD), lambda i: (0, i, 0, 0)),
            pl.BlockSpec((RR, GB, 128, DD), lambda i: (0, i, 0, 0)),
            pl.BlockSpec((RR, GB, 128), lambda i: (0, i, 0)),
            pl.BlockSpec((RR, 1, DD), lambda i: (0, 0, 0)),
            pl.BlockSpec((1, DD), lambda i: (0, 0)),
        ],
        out_specs=pl.BlockSpec((GB, 128, DD), lambda i: (i, 0, 0)),
        out_shape=jax.ShapeDtypeStruct((GG, 128, DD), jnp.float32),
    )(m2, t2, dis, b2, wa)


def kernel(x, edge_index, edge_type, W1, b1, W2, b2, Wa, ba, relation_to_index):
    del ba, relation_to_index
    src = edge_index[0]
    dst = edge_index[1]
    typ = edge_type

    x3 = jnp.pad(x, ((0, NP - NN), (0, 0))).reshape(GG, 128, DD)

    cntp, epk = _sc_counts()(src, dst, typ)
    cntp = cntp.reshape(NC * NS, RR, GG, 128)
    dis, t1 = _tc1(cntp, x3, W1)
    m1 = _sc_msg()(t1.reshape(RR * NP, DD), epk)
    t2 = _tc2(m1.reshape(RR, GG, 128, DD), t1, dis, b1.reshape(RR, 1, DD), W2)
    m2 = _sc_msg()(t2.reshape(RR * NP, DD), epk)
    out = _tc3(m2.reshape(RR, GG, 128, DD), t2, dis, b2.reshape(RR, 1, DD),
               Wa.reshape(1, DD))
    return out.reshape(NP, DD)[:NN]


# speedup vs baseline: 1.0891x; 1.0891x over previous
import functools

import jax
import jax.numpy as jnp
from jax import lax
from jax.experimental import pallas as pl
from jax.experimental.pallas import tpu as pltpu
from jax.experimental.pallas import tpu_sc as plsc

NN = 10000
NP = 10240
EE = 320000
RR = 4
DD = 128
GG = NP // 128
GB = 16
NBLK = GG // GB

NC = 2
NS = 16
EPT = EE // NS
EPT32 = EE // (NC * NS)
KE = 4000
KE2 = 2000
CAP = 20480
NPH = 5120
NRNG = 2
RPT = NPH // NS

def _mesh():
    return plsc.VectorSubcoreMesh(
        core_axis_name="c", subcore_axis_name="s",
        num_cores=NC, num_subcores=NS)


def _sc_counts_body(src_hbm, dst_hbm, typ_hbm, cnt_hbm, epk_hbm,
                    sbuf, dbuf, tbuf, pbuf, acc):
    c = lax.axis_index("c")
    s = lax.axis_index("s")
    wid = s * NC + c
    z16 = jnp.zeros((16,), jnp.float32)
    ones = jnp.ones((16,), jnp.float32)

    def zero(i, _):
        acc[pl.ds(i * 16, 16)] = z16
        return 0
    lax.fori_loop(0, RR * NP // 16, zero, 0)

    def stage(st, _):
        base = wid * EPT32 + st * KE2
        pltpu.sync_copy(src_hbm.at[pl.ds(base, KE2)], sbuf)
        pltpu.sync_copy(dst_hbm.at[pl.ds(base, KE2)], dbuf)
        pltpu.sync_copy(typ_hbm.at[pl.ds(base, KE2)], tbuf)

        def chunk(i, _):
            s16 = sbuf[pl.ds(i * 16, 16)]
            d16 = dbuf[pl.ds(i * 16, 16)]
            t16 = tbuf[pl.ds(i * 16, 16)]
            plsc.addupdate_scatter(acc, [t16 * NP + d16], ones)
            pbuf[pl.ds(i * 16, 16)] = t16 * 268435456 + s16 * 16384 + d16
            return 0
        lax.fori_loop(0, KE2 // 16, chunk, 0)
        pltpu.sync_copy(pbuf, epk_hbm.at[pl.ds(base, KE2)])
        return 0
    lax.fori_loop(0, EPT32 // KE2, stage, 0)
    pltpu.sync_copy(acc, cnt_hbm.at[wid])


@functools.cache
def _sc_counts():
    return pl.kernel(
        _sc_counts_body,
        out_type=(jax.ShapeDtypeStruct((NC * NS, RR * NP), jnp.float32),
                  jax.ShapeDtypeStruct((EE,), jnp.int32)),
        mesh=_mesh(),
        scratch_types=[
            pltpu.VMEM((KE2,), jnp.int32),
            pltpu.VMEM((KE2,), jnp.int32),
            pltpu.VMEM((KE2,), jnp.int32),
            pltpu.VMEM((KE2,), jnp.int32),
            pltpu.VMEM((RR * NP,), jnp.float32),
        ],
        compiler_params=pltpu.CompilerParams(needs_layout_passes=False),
    )


def _sc_msg_body(tab_hbm, epk_hbm, out_hbm,
                 sbuf, prel, qbuf, gidx2, dbuf2, rbuf, zbuf, acc, semA, semB):
    c = lax.axis_index("c")
    s = lax.axis_index("s")
    iota = lax.iota(jnp.int32, 16)
    z16 = jnp.zeros((16,), jnp.float32)

    def zrow(i, _):
        def zcol(k, _):
            zbuf[i, pl.ds(k * 16, 16)] = z16
            return 0
        lax.fori_loop(0, DD // 16, zcol, 0)
        return 0
    lax.fori_loop(0, 16, zrow, 0)

    def one_rel(p, _):
        r = 2 * c + p
        def rstage(st, offr):
            pltpu.sync_copy(epk_hbm.at[pl.ds(s * EPT + st * KE, KE)], sbuf)

            def rchunk(i, offr):
                p16 = sbuf[pl.ds(i * 16, 16)]
                m = (p16 // 268435456) == r
                plsc.store_compressed(prel.at[pl.ds(offr, 16)],
                                      p16 % 268435456, mask=m)
                return offr + jnp.sum(m.astype(jnp.int32))
            return lax.fori_loop(0, KE // 16, rchunk, offr)
        offr = lax.fori_loop(0, EPT // KE, rstage, 0)
        prel[pl.ds(offr, 16)] = jnp.full((16,), 268435455, jnp.int32)
        nrc = (offr + 15) // 16

        def one_rng(h, _):
            lo = h * NPH
            def zacc(k, _):
                pltpu.sync_copy(zbuf, acc.at[pl.ds(s * RPT + k * 16, 16)])
                return 0
            lax.fori_loop(0, RPT // 16, zacc, 0)
            plsc.subcore_barrier()

            def qchunk(i, off):
                q16 = prel[pl.ds(i * 16, 16)]
                d16 = q16 % 16384
                s16 = q16 // 16384
                dl = d16 - lo
                m = (dl >= 0) & (dl < NPH)
                plsc.store_compressed(qbuf.at[pl.ds(off, 16)],
                                      (r * NP + s16) * 8192 + dl, mask=m)
                return off + jnp.sum(m.astype(jnp.int32))
            off = lax.fori_loop(0, nrc, qchunk, 0)

            def pad(k, _):
                qbuf[pl.ds(off + k * 16, 16)] = (
                    (r * NP + NN + (k % 4) * 16 + iota) * 8192
                    + (k % 4) * 16 + iota)
                return 0
            lax.fori_loop(0, 16, pad, 0)
            nbp = ((off + 255) // 256) * 2

            def unpack(j, b):
                def cp16(k, _):
                    v = qbuf[pl.ds(j * 128 + k * 16, 16)]
                    gidx2[b, pl.ds(k * 16, 16)] = v // 8192
                    dbuf2[b, pl.ds(k * 16, 16)] = v % 8192
                    return 0
                lax.fori_loop(0, 8, cp16, 0)

            def fire(j, b, sem):
                unpack(j, b)
                pltpu.async_copy(tab_hbm.at[gidx2.at[b]], rbuf.at[b], sem)

            @pl.when(nbp > 0)
            def _prologue():
                fire(0, 0, semA)
                fire(1, 1, semB)

            def pair(jj, _):
                j0 = 2 * jj
                pltpu.make_async_copy(
                    tab_hbm.at[gidx2.at[0]], rbuf.at[0], semA).wait()
                pltpu.sync_copy(rbuf.at[0], acc.at[dbuf2.at[0]], add=True)

                @pl.when(j0 + 2 < nbp)
                def _():
                    fire(j0 + 2, 0, semA)
                pltpu.make_async_copy(
                    tab_hbm.at[gidx2.at[1]], rbuf.at[1], semB).wait()
                pltpu.sync_copy(rbuf.at[1], acc.at[dbuf2.at[1]], add=True)

                @pl.when(j0 + 3 < nbp)
                def _():
                    fire(j0 + 3, 1, semB)
                return 0
            lax.fori_loop(0, nbp // 2, pair, 0)
            plsc.subcore_barrier()

            grow = lo + s * RPT
            @pl.when(grow + RPT <= NP)
            def _writeback():
                pltpu.sync_copy(acc.at[pl.ds(s * RPT, RPT)],
                                out_hbm.at[pl.ds(r * NP + grow, RPT)])
            plsc.subcore_barrier()
            return 0
        lax.fori_loop(0, NRNG, one_rng, 0)
        return 0
    lax.fori_loop(0, 2, one_rel, 0)


@functools.cache
def _sc_msg():
    return pl.kernel(
        _sc_msg_body,
        out_type=jax.ShapeDtypeStruct((RR * NP, DD), jnp.float32),
        mesh=_mesh(),
        scratch_types=[
            pltpu.VMEM((KE,), jnp.int32),
            pltpu.VMEM((CAP,), jnp.int32),
            pltpu.VMEM((CAP,), jnp.int32),
            pltpu.VMEM((2, 128), jnp.int32),
            pltpu.VMEM((2, 128), jnp.int32),
            pltpu.VMEM((2, 128, DD), jnp.float32),
            pltpu.VMEM((16, DD), jnp.float32),
            pltpu.MemorySpace.VMEM_SHARED((NPH, DD), jnp.float32),
            pltpu.SemaphoreType.DMA,
            pltpu.SemaphoreType.DMA,
        ],
        compiler_params=pltpu.CompilerParams(needs_layout_passes=False),
    )


def _tc1_body(cntp_ref, x_ref, w1_ref, dis_ref, t1_ref):
    deg = jnp.sum(cntp_ref[:, 0], axis=0) + 1.0
    d = lax.rsqrt(deg)
    dis_ref[0] = d
    h = jnp.dot(jnp.reshape(x_ref[...], (GB * 128, DD)), w1_ref[0],
                preferred_element_type=jnp.float32)
    t1_ref[0] = jnp.reshape(h, (GB, 128, DD)) * d[:, :, None]


def _tc2_body(m1_ref, t1_ref, dis_ref, b1_ref, w2_ref, t2_ref):
    d = dis_ref[0]
    pre = (m1_ref[0] + t1_ref[0]) * d[:, :, None] + b1_ref[0, 0][None, None, :]
    h = jnp.maximum(pre, 0.0)
    h2 = jnp.dot(jnp.reshape(h, (GB * 128, DD)), w2_ref[0],
                 preferred_element_type=jnp.float32)
    i = pl.program_id(1)
    nid3 = (i * GB * 128
            + lax.broadcasted_iota(jnp.int32, (GB, 128, DD), 0) * 128
            + lax.broadcasted_iota(jnp.int32, (GB, 128, DD), 1))
    t2 = jnp.reshape(h2, (GB, 128, DD)) * d[:, :, None]
    t2_ref[0] = jnp.where(nid3 < NN, t2, 0.0)


def _tc3_body(m2_ref, t2_ref, dis_ref, b2_ref, wa_ref, out_ref):
    d = dis_ref[...]
    emb = ((m2_ref[...] + t2_ref[...]) * d[..., None]
           + b2_ref[:, 0][:, None, None, :])
    wa = wa_ref[0]
    logits = jnp.sum(emb * wa[None, None, None, :], axis=-1)
    mx = jnp.max(logits, axis=0)
    ex = jnp.exp(logits - mx[None])
    att = ex / jnp.sum(ex, axis=0)[None]
    out_ref[...] = jnp.sum(emb * att[..., None], axis=0)


def _tc1(cntp, x3, W1):
    return pl.pallas_call(
        _tc1_body,
        grid=(RR, NBLK),
        in_specs=[
            pl.BlockSpec((NC * NS, 1, GB, 128), lambda r, i: (0, r, i, 0)),
            pl.BlockSpec((GB, 128, DD), lambda r, i: (i, 0, 0)),
            pl.BlockSpec((1, DD, DD), lambda r, i: (r, 0, 0)),
        ],
        out_specs=[
            pl.BlockSpec((1, GB, 128), lambda r, i: (r, i, 0)),
            pl.BlockSpec((1, GB, 128, DD), lambda r, i: (r, i, 0, 0)),
        ],
        out_shape=[
            jax.ShapeDtypeStruct((RR, GG, 128), jnp.float32),
            jax.ShapeDtypeStruct((RR, GG, 128, DD), jnp.float32),
        ],
    )(cntp, x3, W1)


def _tc2(m1, t1, dis, b1, W2):
    return pl.pallas_call(
        _tc2_body,
        grid=(RR, NBLK),
        in_specs=[
            pl.BlockSpec((1, GB, 128, DD), lambda r, i: (r, i, 0, 0)),
            pl.BlockSpec((1, GB, 128, DD), lambda r, i: (r, i, 0, 0)),
            pl.BlockSpec((1, GB, 128), lambda r, i: (r, i, 0)),
            pl.BlockSpec((1, 1, DD), lambda r, i: (r, 0, 0)),
            pl.BlockSpec((1, DD, DD), lambda r, i: (r, 0, 0)),
        ],
        out_specs=pl.BlockSpec((1, GB, 128, DD), lambda r, i: (r, i, 0, 0)),
        out_shape=jax.ShapeDtypeStruct((RR, GG, 128, DD), jnp.float32),
    )(m1, t1, dis, b1, W2)


def _tc3(m2, t2, dis, b2, wa):
    return pl.pallas_call(
        _tc3_body,
        grid=(NBLK,),
        in_specs=[
            pl.BlockSpec((RR, GB, 128, DD), lambda i: (0, i, 0, 0)),
            pl.BlockSpec((RR, GB, 128, DD), lambda i: (0, i, 0, 0)),
            pl.BlockSpec((RR, GB, 128), lambda i: (0, i, 0)),
            pl.BlockSpec((RR, 1, DD), lambda i: (0, 0, 0)),
            pl.BlockSpec((1, DD), lambda i: (0, 0)),
        ],
        out_specs=pl.BlockSpec((GB, 128, DD), lambda i: (i, 0, 0)),
        out_shape=jax.ShapeDtypeStruct((GG, 128, DD), jnp.float32),
    )(m2, t2, dis, b2, wa)


def kernel(x, edge_index, edge_type, W1, b1, W2, b2, Wa, ba, relation_to_index):
    del ba, relation_to_index
    src = edge_index[0]
    dst = edge_index[1]
    typ = edge_type

    x3 = jnp.pad(x, ((0, NP - NN), (0, 0))).reshape(GG, 128, DD)

    cntp, epk = _sc_counts()(src, dst, typ)
    cntp = cntp.reshape(NC * NS, RR, GG, 128)
    dis, t1 = _tc1(cntp, x3, W1)
    m1 = _sc_msg()(t1.reshape(RR * NP, DD), epk)
    t2 = _tc2(m1.reshape(RR, GG, 128, DD), t1, dis, b1.reshape(RR, 1, DD), W2)
    m2 = _sc_msg()(t2.reshape(RR * NP, DD), epk)
    out = _tc3(m2.reshape(RR, GG, 128, DD), t2, dis, b2.reshape(RR, 1, DD),
               Wa.reshape(1, DD))
    return out.reshape(NP, DD)[:NN]
